# baseline (speedup 1.0000x reference)
import functools

import jax
import jax.numpy as jnp
from jax import lax
from jax.experimental import pallas as pl
from jax.experimental.pallas import tpu as pltpu
from jax.experimental.pallas import tpu_sc as plsc

_K = 8
_DD = 16
_D = 128
_N = 10000
_M = 320000
_NITER = 6

_NW = 32
_NPAD = 10240
_EPW = 10240
_MPAD = _NW * _EPW
_C = 80
_NCH = _EPW // _C
_RPT = _NPAD // 16


def _sc_edge_pass_body(u_hbm, xn_hbm, edges_hbm, out_hbm,
                       acc_sh, idx_b, z_b, ut_b,
                       sz0, sz1, su0, su1):
  cid = lax.axis_index("c")
  sid = lax.axis_index("s")
  wid = sid * 2 + cid

  zvec = jnp.zeros((16,), jnp.float32)

  def _zrow(i, _):
    for j in range(_D // 16):
      ut_b[i, pl.ds(j * 16, 16)] = zvec
    return 0

  lax.fori_loop(0, 2 * _C, _zrow, 0)
  for b in range(_RPT // (2 * _C)):
    pltpu.sync_copy(ut_b, acc_sh.at[pl.ds(sid * _RPT + b * 2 * _C, 2 * _C)])
  plsc.subcore_barrier()

  lane = lax.broadcasted_iota(jnp.int32, (16,), 0)
  szs = (sz0, sz1)
  sus = (su0, su1)

  def _fetch(cj, sl):
    pltpu.sync_copy(edges_hbm.at[wid, cj], idx_b.at[sl])
    pltpu.async_copy(xn_hbm.at[idx_b.at[sl, 0]],
                     z_b.at[pl.ds(sl * _C, _C)], szs[sl])
    pltpu.async_copy(u_hbm.at[idx_b.at[sl, 1]],
                     ut_b.at[pl.ds(sl * _C, _C)], sus[sl])

  def _wait(sl):
    pltpu.make_async_copy(xn_hbm.at[idx_b.at[sl, 0]],
                          z_b.at[pl.ds(sl * _C, _C)], szs[sl]).wait()
    pltpu.make_async_copy(u_hbm.at[idx_b.at[sl, 1]],
                          ut_b.at[pl.ds(sl * _C, _C)], sus[sl]).wait()

  def _scatter(sl):
    pltpu.sync_copy(ut_b.at[pl.ds(sl * _C, _C)],
                    acc_sh.at[idx_b.at[sl, 1]], add=True)

  rots = [jnp.bitwise_and(lane + j, 15) for j in range(_DD)]

  def _compute(sl):
    soff = sl * _C

    def _group(g, _):
      rows = lane + (g * 16 + soff)
      ps = []
      for k in range(_K):
        accs = [None] * 4
        for j in range(_DD):
          col = rots[j] + (k * _DD)
          zz = plsc.load_gather(z_b, [rows, col])
          uu = plsc.load_gather(ut_b, [rows, col])
          prod = zz * uu
          a = j % 4
          accs[a] = prod if accs[a] is None else accs[a] + prod
        ps.append((accs[0] + accs[1]) + (accs[2] + accs[3]))
      es = [jnp.exp(p) for p in ps]
      s = (es[0] + es[1]) + (es[2] + es[3])
      s = s + ((es[4] + es[5]) + (es[6] + es[7]))
      inv = 1.0 / s
      for k in range(_K):
        w = es[k] * inv
        for j in range(_DD):
          col = rots[j] + (k * _DD)
          zz = plsc.load_gather(z_b, [rows, col])
          plsc.store_scatter(ut_b, [rows, col], zz * w)
      return 0

    lax.fori_loop(0, _C // 16, _group, 0)

  @pl.when(jnp.int32(_NCH) > 0)
  def _():
    _fetch(0, 0)

  def _it(c, _):
    sl = lax.rem(c, 2)

    @pl.when(c + 1 < _NCH)
    def _():
      @pl.when(sl == 0)
      def _():
        _fetch(c + 1, 1)

      @pl.when(sl == 1)
      def _():
        _fetch(c + 1, 0)

    @pl.when(sl == 0)
    def _():
      _wait(0)

    @pl.when(sl == 1)
    def _():
      _wait(1)

    _compute(sl)

    @pl.when(sl == 0)
    def _():
      _scatter(0)

    @pl.when(sl == 1)
    def _():
      _scatter(1)

    return 0

  lax.fori_loop(0, _NCH, _it, 0)

  plsc.subcore_barrier()
  pltpu.sync_copy(acc_sh.at[pl.ds(sid * _RPT, _RPT)],
                  out_hbm.at[cid, pl.ds(sid * _RPT, _RPT)])


_sc_edge_pass = pl.kernel(
    _sc_edge_pass_body,
    out_type=jax.ShapeDtypeStruct((2, _NPAD, _D), jnp.float32),
    mesh=plsc.VectorSubcoreMesh(core_axis_name="c", subcore_axis_name="s"),
    scratch_types=[
        pltpu.VMEM_SHARED((_NPAD, _D), jnp.float32),
        pltpu.VMEM((2, 2, _C), jnp.int32),
        pltpu.VMEM((2 * _C, _D), jnp.float32),
        pltpu.VMEM((2 * _C, _D), jnp.float32),
        pltpu.SemaphoreType.DMA,
        pltpu.SemaphoreType.DMA,
        pltpu.SemaphoreType.DMA,
        pltpu.SemaphoreType.DMA,
    ],
    compiler_params=pltpu.CompilerParams(needs_layout_passes=False),
    name="disen_edge_pass",
)


def _norm_chunks(v):
  parts = []
  for k in range(_K):
    s = v[:, k * _DD:(k + 1) * _DD]
    n = jnp.sqrt(jnp.sum(s * s, axis=1, keepdims=True))
    parts.append(s / jnp.maximum(n, 1e-12))
  return jnp.concatenate(parts, axis=1)


def _tc_init_body(x_ref, o_ref):
  o_ref[...] = _norm_chunks(x_ref[...])


def _tc_comb_body(p0_ref, p1_ref, xn_ref, o_ref):
  o_ref[...] = _norm_chunks(p0_ref[...] + p1_ref[...] + xn_ref[...])


_TCB = 256
_spec = pl.BlockSpec((_TCB, _D), lambda i: (i, 0))

_tc_init = pl.pallas_call(
    _tc_init_body,
    grid=(_NPAD // _TCB,),
    in_specs=[_spec],
    out_specs=_spec,
    out_shape=jax.ShapeDtypeStruct((_NPAD, _D), jnp.float32),
)

_tc_comb = pl.pallas_call(
    _tc_comb_body,
    grid=(_NPAD // _TCB,),
    in_specs=[_spec, _spec, _spec],
    out_specs=_spec,
    out_shape=jax.ShapeDtypeStruct((_NPAD, _D), jnp.float32),
)


@jax.jit
def kernel(x, edge_index):
  x = x.astype(jnp.float32)
  xp = jnp.pad(x, ((0, _NPAD - _N), (0, 0)))
  xn = _tc_init(xp)

  npad_e = _MPAD - _M
  pad_idx = _N + (jnp.arange(npad_e, dtype=jnp.int32) % (_NPAD - _N))
  srcp = jnp.concatenate([edge_index[0].astype(jnp.int32), pad_idx])
  trgp = jnp.concatenate([edge_index[1].astype(jnp.int32), pad_idx])
  edges = jnp.stack(
      [srcp.reshape(_NW, _NCH, _C), trgp.reshape(_NW, _NCH, _C)], axis=2)

  u = xn
  for _ in range(_NITER):
    parts = _sc_edge_pass(u, xn, edges)
    u = _tc_comb(parts[0], parts[1], xn)
  return u[:_N]


# speedup vs baseline: 1.0002x; 1.0002x over previous
import functools

import jax
import jax.numpy as jnp
from jax import lax
from jax.experimental import pallas as pl
from jax.experimental.pallas import tpu as pltpu
from jax.experimental.pallas import tpu_sc as plsc

_K = 8
_DD = 16
_D = 128
_N = 10000
_M = 320000
_NITER = 6

_NW = 32
_NPAD = 10240
_EPW = 10240
_MPAD = _NW * _EPW
_C = 80
_NCH = _EPW // _C
_RPT = _NPAD // 16


def _sc_edge_pass_body(u_hbm, xn_hbm, edges_hbm, out_hbm,
                       acc_sh, idx_b, z_b, ut_b,
                       sz0, sz1, su0, su1, si0, si1):
  cid = lax.axis_index("c")
  sid = lax.axis_index("s")
  wid = sid * 2 + cid

  zvec = jnp.zeros((16,), jnp.float32)

  def _zrow(i, _):
    for j in range(_D // 16):
      ut_b[i, pl.ds(j * 16, 16)] = zvec
    return 0

  lax.fori_loop(0, 2 * _C, _zrow, 0)
  for b in range(_RPT // (2 * _C)):
    pltpu.sync_copy(ut_b, acc_sh.at[pl.ds(sid * _RPT + b * 2 * _C, 2 * _C)])
  plsc.subcore_barrier()

  lane = lax.broadcasted_iota(jnp.int32, (16,), 0)
  szs = (sz0, sz1)
  sus = (su0, su1)
  sis = (si0, si1)

  def _idx_start(cj, sl):
    pltpu.async_copy(edges_hbm.at[wid, cj], idx_b.at[sl], sis[sl])

  def _idx_wait(cj, sl):
    pltpu.make_async_copy(edges_hbm.at[wid, cj], idx_b.at[sl],
                          sis[sl]).wait()

  def _fetch(cj, sl):
    _idx_wait(cj, sl)
    pltpu.async_copy(xn_hbm.at[idx_b.at[sl, 0]],
                     z_b.at[pl.ds(sl * _C, _C)], szs[sl])
    pltpu.async_copy(u_hbm.at[idx_b.at[sl, 1]],
                     ut_b.at[pl.ds(sl * _C, _C)], sus[sl])

  def _wait(sl):
    pltpu.make_async_copy(xn_hbm.at[idx_b.at[sl, 0]],
                          z_b.at[pl.ds(sl * _C, _C)], szs[sl]).wait()
    pltpu.make_async_copy(u_hbm.at[idx_b.at[sl, 1]],
                          ut_b.at[pl.ds(sl * _C, _C)], sus[sl]).wait()

  def _scatter(sl):
    pltpu.sync_copy(ut_b.at[pl.ds(sl * _C, _C)],
                    acc_sh.at[idx_b.at[sl, 1]], add=True)

  rots = [jnp.bitwise_and(lane + j, 15) for j in range(_DD)]

  def _compute(sl):
    soff = sl * _C

    def _group(g, _):
      rows = lane + (g * 16 + soff)
      ps = []
      for k in range(_K):
        accs = [None] * 4
        for j in range(_DD):
          col = rots[j] + (k * _DD)
          zz = plsc.load_gather(z_b, [rows, col])
          uu = plsc.load_gather(ut_b, [rows, col])
          prod = zz * uu
          a = j % 4
          accs[a] = prod if accs[a] is None else accs[a] + prod
        ps.append((accs[0] + accs[1]) + (accs[2] + accs[3]))
      es = [jnp.exp(p) for p in ps]
      s = (es[0] + es[1]) + (es[2] + es[3])
      s = s + ((es[4] + es[5]) + (es[6] + es[7]))
      inv = 1.0 / s
      for k in range(_K):
        w = es[k] * inv
        for j in range(_DD):
          col = rots[j] + (k * _DD)
          zz = plsc.load_gather(z_b, [rows, col])
          plsc.store_scatter(ut_b, [rows, col], zz * w)
      return 0

    lax.fori_loop(0, _C // 16, _group, 0)

  _idx_start(0, 0)
  _fetch(0, 0)
  _idx_start(1, 1)

  def _it(c, _):
    sl = lax.rem(c, 2)

    @pl.when(c + 1 < _NCH)
    def _():
      @pl.when(sl == 0)
      def _():
        _fetch(c + 1, 1)

      @pl.when(sl == 1)
      def _():
        _fetch(c + 1, 0)

    @pl.when(sl == 0)
    def _():
      _wait(0)

    @pl.when(sl == 1)
    def _():
      _wait(1)

    _compute(sl)

    @pl.when(sl == 0)
    def _():
      _scatter(0)

      @pl.when(c + 2 < _NCH)
      def _():
        _idx_start(c + 2, 0)

    @pl.when(sl == 1)
    def _():
      _scatter(1)

      @pl.when(c + 2 < _NCH)
      def _():
        _idx_start(c + 2, 1)

    return 0

  lax.fori_loop(0, _NCH, _it, 0)

  plsc.subcore_barrier()
  pltpu.sync_copy(acc_sh.at[pl.ds(sid * _RPT, _RPT)],
                  out_hbm.at[cid, pl.ds(sid * _RPT, _RPT)])


_sc_edge_pass = pl.kernel(
    _sc_edge_pass_body,
    out_type=jax.ShapeDtypeStruct((2, _NPAD, _D), jnp.float32),
    mesh=plsc.VectorSubcoreMesh(core_axis_name="c", subcore_axis_name="s"),
    scratch_types=[
        pltpu.VMEM_SHARED((_NPAD, _D), jnp.float32),
        pltpu.VMEM((2, 2, _C), jnp.int32),
        pltpu.VMEM((2 * _C, _D), jnp.float32),
        pltpu.VMEM((2 * _C, _D), jnp.float32),
        pltpu.SemaphoreType.DMA,
        pltpu.SemaphoreType.DMA,
        pltpu.SemaphoreType.DMA,
        pltpu.SemaphoreType.DMA,
        pltpu.SemaphoreType.DMA,
        pltpu.SemaphoreType.DMA,
    ],
    compiler_params=pltpu.CompilerParams(needs_layout_passes=False),
    name="disen_edge_pass",
)


def _norm_chunks(v):
  parts = []
  for k in range(_K):
    s = v[:, k * _DD:(k + 1) * _DD]
    n = jnp.sqrt(jnp.sum(s * s, axis=1, keepdims=True))
    parts.append(s / jnp.maximum(n, 1e-12))
  return jnp.concatenate(parts, axis=1)


def _tc_init_body(x_ref, o_ref):
  o_ref[...] = _norm_chunks(x_ref[...])


def _tc_comb_body(p0_ref, p1_ref, xn_ref, o_ref):
  o_ref[...] = _norm_chunks(p0_ref[...] + p1_ref[...] + xn_ref[...])


_TCB = 256
_spec = pl.BlockSpec((_TCB, _D), lambda i: (i, 0))

_tc_init = pl.pallas_call(
    _tc_init_body,
    grid=(_NPAD // _TCB,),
    in_specs=[_spec],
    out_specs=_spec,
    out_shape=jax.ShapeDtypeStruct((_NPAD, _D), jnp.float32),
)

_tc_comb = pl.pallas_call(
    _tc_comb_body,
    grid=(_NPAD // _TCB,),
    in_specs=[_spec, _spec, _spec],
    out_specs=_spec,
    out_shape=jax.ShapeDtypeStruct((_NPAD, _D), jnp.float32),
)


@jax.jit
def kernel(x, edge_index):
  x = x.astype(jnp.float32)
  xp = jnp.pad(x, ((0, _NPAD - _N), (0, 0)))
  xn = _tc_init(xp)

  npad_e = _MPAD - _M
  pad_idx = _N + (jnp.arange(npad_e, dtype=jnp.int32) % (_NPAD - _N))
  srcp = jnp.concatenate([edge_index[0].astype(jnp.int32), pad_idx])
  trgp = jnp.concatenate([edge_index[1].astype(jnp.int32), pad_idx])
  edges = jnp.stack(
      [srcp.reshape(_NW, _NCH, _C), trgp.reshape(_NW, _NCH, _C)], axis=2)

  u = xn
  for _ in range(_NITER):
    parts = _sc_edge_pass(u, xn, edges)
    u = _tc_comb(parts[0], parts[1], xn)
  return u[:_N]
